# baseline (speedup 1.0000x reference)
import functools

import jax
import jax.numpy as jnp
from jax import lax
from jax.experimental import pallas as pl
from jax.experimental.pallas import tpu as pltpu
from jax.experimental.pallas import tpu_sc as plsc


@functools.lru_cache(maxsize=None)
def _build(seq_len: int, batch: int, vocab: int, dim: int):
    info = plsc.get_sparse_core_info()
    nc, ns, lanes = info.num_cores, info.num_subcores, info.num_lanes
    nw = nc * ns
    bpw = batch // nw
    nblk = bpw // 128
    nslot = 16
    brows = 1
    nbatch = bpw // brows
    assert bpw % (nslot * brows) == 0 and dim == 2 * lanes

    mesh = plsc.VectorSubcoreMesh(core_axis_name="c", subcore_axis_name="s")

    @functools.partial(
        pl.kernel,
        mesh=mesh,
        compiler_params=pltpu.CompilerParams(
            needs_layout_passes=False, disable_bounds_checks=True),
        out_type=jax.ShapeDtypeStruct((dim, batch), jnp.float32),
        scratch_types=[
            pltpu.VMEM((bpw,), jnp.int32),
            pltpu.VMEM((seq_len, 128), jnp.int32),
            pltpu.VMEM((bpw + lanes,), jnp.int32),
            pltpu.VMEM((nslot, brows, dim, 128), jnp.float32),
            pltpu.VMEM((dim, bpw), jnp.float32),
            pltpu.SemaphoreType.DMA,
        ] + [pltpu.SemaphoreType.DMA for _ in range(nslot)],
    )
    def k(items_hbm, seq_hbm, tabt_hbm, out_hbm, sl_v, ib_v, li_v,
          ring_v, out_v, sem_i, *sems):
        def li_at(t):
            return li_v[pl.ds(t, lanes)][0]
        wid = lax.axis_index("s") * nc + lax.axis_index("c")
        base = wid * bpw
        pltpu.sync_copy(seq_hbm.at[pl.ds(base, bpw)], sl_v)
        iota = lax.iota(jnp.int32, lanes)

        for j in range(nblk):
            pltpu.async_copy(
                items_hbm.at[:, pl.ds(base + j * 128, 128)], ib_v, sem_i
            ).wait()
            for g in range(128 // lanes):
                o = j * 128 + g * lanes
                rows = sl_v[pl.ds(o, lanes)] - 1
                vals = plsc.load_gather(ib_v, [rows, g * lanes + iota])
                li_v[pl.ds(o, lanes)] = vals

        def fire(t, slot, k_row):
            r = li_at(t)
            rb = pl.multiple_of((r >> 7) * 128, 128)
            pltpu.async_copy(
                tabt_hbm.at[:, pl.ds(rb, 128)],
                ring_v.at[slot, k_row], sems[slot],
            )

        def extract(t, slot, k_row):
            r = li_at(t)
            c = jnp.full((lanes,), r & 127, jnp.int32)
            s_v = jnp.full((lanes,), slot, jnp.int32)
            k_v = jnp.full((lanes,), k_row, jnp.int32)
            t_v = jnp.full((lanes,), t, jnp.int32)
            for h in range(dim // lanes):
                vals = plsc.load_gather(
                    ring_v, [s_v, k_v, h * lanes + iota, c])
                plsc.store_scatter(out_v, [h * lanes + iota, t_v], vals)

        for b in range(nslot - 1):
            for k_row in range(brows):
                fire(b * brows + k_row, b, k_row)

        def body(it, carry):
            for s in range(nslot):
                bid = it * nslot + s
                t0 = bid * brows
                for k_row in range(brows):
                    pltpu.make_async_copy(
                        tabt_hbm.at[:, pl.ds(0, 128)],
                        ring_v.at[s, k_row], sems[s],
                    ).wait()
                nb = bid + nslot - 1

                @pl.when(nb < nbatch)
                def _():
                    for k_row in range(brows):
                        fire(nb * brows + k_row, (s + nslot - 1) % nslot,
                             k_row)

                for k_row in range(brows):
                    extract(t0 + k_row, s, k_row)
            return carry

        lax.fori_loop(0, nbatch // nslot, body, 0)
        pltpu.sync_copy(out_v, out_hbm.at[:, pl.ds(base, bpw)])

    return k


def kernel(items, seq_lens, table):
    seq_len, batch = items.shape
    vocab, dim = table.shape
    seq_lens = seq_lens.astype(jnp.int32)
    k = _build(seq_len, batch, vocab, dim)
    out = k(items.astype(jnp.int32), seq_lens, table.T)
    return out.T


# speedup vs baseline: 1.0649x; 1.0649x over previous
import functools

import jax
import jax.numpy as jnp
from jax import lax
from jax.experimental import pallas as pl
from jax.experimental.pallas import tpu as pltpu
from jax.experimental.pallas import tpu_sc as plsc


@functools.lru_cache(maxsize=None)
def _build(seq_len: int, batch: int, vocab: int, dim: int):
    info = plsc.get_sparse_core_info()
    nc, ns, lanes = info.num_cores, info.num_subcores, info.num_lanes
    nw = nc * ns
    bpw = batch // nw
    nblk = bpw // 128
    nslot = 8
    brows = 2
    nbatch = bpw // brows
    assert bpw % (nslot * brows) == 0 and dim == 2 * lanes

    mesh = plsc.VectorSubcoreMesh(core_axis_name="c", subcore_axis_name="s")

    @functools.partial(
        pl.kernel,
        mesh=mesh,
        compiler_params=pltpu.CompilerParams(
            needs_layout_passes=False, disable_bounds_checks=True),
        out_type=jax.ShapeDtypeStruct((dim, batch), jnp.float32),
        scratch_types=[
            pltpu.VMEM((bpw,), jnp.int32),
            pltpu.VMEM((seq_len, 128), jnp.int32),
            pltpu.VMEM((bpw + lanes,), jnp.int32),
            pltpu.VMEM((nslot, brows, dim, 128), jnp.float32),
            pltpu.VMEM((dim, bpw), jnp.float32),
            pltpu.SemaphoreType.DMA,
        ] + [pltpu.SemaphoreType.DMA for _ in range(nslot)],
    )
    def k(items_hbm, seq_hbm, tabt_hbm, out_hbm, sl_v, ib_v, li_v,
          ring_v, out_v, sem_i, *sems):
        def li_at(t):
            return li_v[pl.ds(t, lanes)][0]
        wid = lax.axis_index("s") * nc + lax.axis_index("c")
        base = wid * bpw
        pltpu.sync_copy(seq_hbm.at[pl.ds(base, bpw)], sl_v)
        iota = lax.iota(jnp.int32, lanes)

        for j in range(nblk):
            pltpu.async_copy(
                items_hbm.at[:, pl.ds(base + j * 128, 128)], ib_v, sem_i
            ).wait()
            for g in range(128 // lanes):
                o = j * 128 + g * lanes
                rows = sl_v[pl.ds(o, lanes)] - 1
                vals = plsc.load_gather(ib_v, [rows, g * lanes + iota])
                li_v[pl.ds(o, lanes)] = vals

        def fire(t, slot, k_row):
            r = li_at(t)
            rb = pl.multiple_of((r >> 7) * 128, 128)
            pltpu.async_copy(
                tabt_hbm.at[:, pl.ds(rb, 128)],
                ring_v.at[slot, k_row], sems[slot],
            )

        def extract(t, slot, k_row):
            r = li_at(t)
            c = jnp.full((lanes,), r & 127, jnp.int32)
            s_v = jnp.full((lanes,), slot, jnp.int32)
            k_v = jnp.full((lanes,), k_row, jnp.int32)
            t_v = jnp.full((lanes,), t, jnp.int32)
            for h in range(dim // lanes):
                vals = plsc.load_gather(
                    ring_v, [s_v, k_v, h * lanes + iota, c])
                plsc.store_scatter(out_v, [h * lanes + iota, t_v], vals)

        for b in range(nslot - 1):
            for k_row in range(brows):
                fire(b * brows + k_row, b, k_row)

        def body(it, carry):
            for s in range(nslot):
                bid = it * nslot + s
                t0 = bid * brows
                for k_row in range(brows):
                    pltpu.make_async_copy(
                        tabt_hbm.at[:, pl.ds(0, 128)],
                        ring_v.at[s, k_row], sems[s],
                    ).wait()
                nb = bid + nslot - 1

                @pl.when(nb < nbatch)
                def _():
                    for k_row in range(brows):
                        fire(nb * brows + k_row, (s + nslot - 1) % nslot,
                             k_row)

                for k_row in range(brows):
                    extract(t0 + k_row, s, k_row)
            return carry

        lax.fori_loop(0, nbatch // nslot, body, 0)
        pltpu.sync_copy(out_v, out_hbm.at[:, pl.ds(base, bpw)])

    return k


def kernel(items, seq_lens, table):
    seq_len, batch = items.shape
    vocab, dim = table.shape
    seq_lens = seq_lens.astype(jnp.int32)
    k = _build(seq_len, batch, vocab, dim)
    out = k(items.astype(jnp.int32), seq_lens, table.T)
    return out.T


# speedup vs baseline: 1.0699x; 1.0047x over previous
import functools

import jax
import jax.numpy as jnp
from jax import lax
from jax.experimental import pallas as pl
from jax.experimental.pallas import tpu as pltpu
from jax.experimental.pallas import tpu_sc as plsc


@functools.lru_cache(maxsize=None)
def _build(seq_len: int, batch: int, vocab: int, dim: int):
    info = plsc.get_sparse_core_info()
    nc, ns, lanes = info.num_cores, info.num_subcores, info.num_lanes
    nw = nc * ns
    bpw = batch // nw
    nblk = bpw // 128
    nslot = 8
    brows = 2
    nbatch = bpw // brows
    assert bpw % (nslot * brows) == 0 and dim == 2 * lanes

    mesh = plsc.VectorSubcoreMesh(core_axis_name="c", subcore_axis_name="s")

    @functools.partial(
        pl.kernel,
        mesh=mesh,
        compiler_params=pltpu.CompilerParams(
            needs_layout_passes=False, disable_bounds_checks=True),
        out_type=jax.ShapeDtypeStruct((dim, batch), jnp.float32),
        scratch_types=[
            pltpu.VMEM((bpw,), jnp.int32),
            pltpu.VMEM((seq_len, 128), jnp.int32),
            pltpu.VMEM((bpw + lanes,), jnp.int32),
            pltpu.VMEM((nslot, brows, dim, 128), jnp.float32),
            pltpu.VMEM((dim, bpw), jnp.float32),
            pltpu.SemaphoreType.DMA,
        ] + [pltpu.SemaphoreType.DMA for _ in range(nslot)],
    )
    def k(items_hbm, seq_hbm, tabt_hbm, out_hbm, sl_v, ib_v, li_v,
          ring_v, out_v, sem_i, *sems):
        def li_at(t):
            return li_v[pl.ds(t, lanes)][0]
        wid = lax.axis_index("s") * nc + lax.axis_index("c")
        base = wid * bpw
        pltpu.sync_copy(seq_hbm.at[pl.ds(base, bpw)], sl_v)
        iota = lax.iota(jnp.int32, lanes)

        def items_block(j):
            pltpu.async_copy(
                items_hbm.at[:, pl.ds(base + j * 128, 128)], ib_v, sem_i
            ).wait()
            for g in range(128 // lanes):
                o = j * 128 + g * lanes
                rows = sl_v[pl.ds(o, lanes)] - 1
                vals = plsc.load_gather(ib_v, [rows, g * lanes + iota])
                li_v[pl.ds(o, lanes)] = vals

        items_block(0)

        def fire(t, slot, k_row):
            r = li_at(t)
            rb = pl.multiple_of((r >> 7) * 128, 128)
            pltpu.async_copy(
                tabt_hbm.at[:, pl.ds(rb, 128)],
                ring_v.at[slot, k_row], sems[slot],
            )

        def extract(t, slot, k_row):
            r = li_at(t)
            c = jnp.full((lanes,), r & 127, jnp.int32)
            s_v = jnp.full((lanes,), slot, jnp.int32)
            k_v = jnp.full((lanes,), k_row, jnp.int32)
            t_v = jnp.full((lanes,), t, jnp.int32)
            for h in range(dim // lanes):
                vals = plsc.load_gather(
                    ring_v, [s_v, k_v, h * lanes + iota, c])
                plsc.store_scatter(out_v, [h * lanes + iota, t_v], vals)

        for b in range(nslot - 1):
            for k_row in range(brows):
                fire(b * brows + k_row, b, k_row)
        for j in range(1, nblk):
            items_block(j)

        def body(it, carry):
            for s in range(nslot):
                bid = it * nslot + s
                t0 = bid * brows
                for k_row in range(brows):
                    pltpu.make_async_copy(
                        tabt_hbm.at[:, pl.ds(0, 128)],
                        ring_v.at[s, k_row], sems[s],
                    ).wait()
                nb = bid + nslot - 1

                @pl.when(nb < nbatch)
                def _():
                    for k_row in range(brows):
                        fire(nb * brows + k_row, (s + nslot - 1) % nslot,
                             k_row)

                for k_row in range(brows):
                    extract(t0 + k_row, s, k_row)
            return carry

        lax.fori_loop(0, nbatch // nslot, body, 0)
        pltpu.sync_copy(out_v, out_hbm.at[:, pl.ds(base, bpw)])

    return k


def kernel(items, seq_lens, table):
    seq_len, batch = items.shape
    vocab, dim = table.shape
    seq_lens = seq_lens.astype(jnp.int32)
    k = _build(seq_len, batch, vocab, dim)
    out = k(items.astype(jnp.int32), seq_lens, table.T)
    return out.T
